# baseline (speedup 1.0000x reference)
import functools
import math

import jax
import jax.numpy as jnp
from jax import lax
from jax.experimental import pallas as pl
from jax.experimental.pallas import tpu as pltpu
from jax.experimental.pallas import tpu_sc as plsc

VOCAB = 100000
DIM = 64
SEQ = 1500
BATCH = 1024
SEQ_PAD = 1504
PACK_ROWS = 768

NC = 2
NS = 16
NW = NC * NS
ROWS_PER_W = BATCH // NW

CHUNKS = ((0, 384), (384, 384), (768, 384), (1152, 348))
CLMAX = 384
IGS = 128


def _pe_table():
    position = jnp.arange(SEQ, dtype=jnp.float32)[:, None]
    div_term = jnp.exp(
        jnp.arange(0, DIM, 2, dtype=jnp.float32) * (-(math.log(10000.0) / DIM))
    )
    ang = position * div_term
    pe = jnp.zeros((SEQ, DIM), dtype=jnp.float32)
    pe = pe.at[:, 0::2].set(jnp.sin(ang))
    pe = pe.at[:, 1::2].set(jnp.cos(ang))
    return pe


@functools.partial(
    pl.kernel,
    mesh=plsc.VectorSubcoreMesh(core_axis_name="c", subcore_axis_name="s"),
    out_type=jax.ShapeDtypeStruct((BATCH, PACK_ROWS, 128), jnp.float32),
    scratch_types=[
        pltpu.VMEM((CLMAX, DIM), jnp.float32),
        pltpu.VMEM((CLMAX,), jnp.int32),
        pltpu.VMEM((CLMAX,), jnp.int32),
        pltpu.VMEM((CLMAX, DIM), jnp.float32),
        pltpu.VMEM((CLMAX, DIM), jnp.float32),
        pltpu.VMEM((CLMAX // 2, 128), jnp.float32),
        pltpu.VMEM((CLMAX // 2, 128), jnp.float32),
        pltpu.SemaphoreType.DMA,
        pltpu.SemaphoreType.DMA,
        pltpu.SemaphoreType.DMA,
        pltpu.SemaphoreType.DMA,
        pltpu.SemaphoreType.DMA,
        pltpu.SemaphoreType.DMA,
    ],
    compiler_params=pltpu.CompilerParams(use_tc_tiling_on_sc=False),
)
def _seq_encode(tok_hbm, pe_hbm, table_hbm, out_hbm,
                pe_v, idx0, idx1, rows0, rows1, pack0, pack1,
                sem_i0, sem_i1, sem_g0, sem_g1, sem_s0, sem_s1):
    wid = lax.axis_index("s") * NC + lax.axis_index("c")

    for off, cl in CHUNKS:
        cl_pad = -(-cl // 8) * 8
        n_g, rem = divmod(cl_pad, IGS)
        qn = cl // 2
        unroll = 8 if qn % 8 == 0 else (4 if qn % 4 == 0 else 2)

        pltpu.sync_copy(pe_hbm.at[pl.ds(off, cl), :], pe_v.at[pl.ds(0, cl), :])

        def fire_gathers(idx_v, rows_v, sem, n_g=n_g, rem=rem):
            handles = []
            for j in range(n_g):
                handles.append(pltpu.async_copy(
                    table_hbm.at[idx_v.at[pl.ds(j * IGS, IGS)]],
                    rows_v.at[pl.ds(j * IGS, IGS), :], sem))
            if rem:
                handles.append(pltpu.async_copy(
                    table_hbm.at[idx_v.at[pl.ds(n_g * IGS, rem)]],
                    rows_v.at[pl.ds(n_g * IGS, rem), :], sem))
            return handles

        def add_pack(rows_v, pack_v, qn=qn, unroll=unroll):
            def body(i, c):
                for u in range(unroll):
                    q = i * unroll + u
                    for h in range(2):
                        for v in range(DIM // 16):
                            src = pl.ds(v * 16, 16)
                            dst = pl.ds(h * DIM + v * 16, 16)
                            pack_v[q, dst] = rows_v[2 * q + h, src] + pe_v[2 * q + h, src]
                return c
            lax.fori_loop(0, qn // unroll, body, 0)

        def drain_store(pack_v, sem, off=off, qn=qn):
            pltpu.make_async_copy(
                pack_v.at[pl.ds(0, qn), :],
                out_hbm.at[0, pl.ds(off // 2, qn), :], sem).wait()

        def pair_body(r2, carry, off=off, cl=cl, cl_pad=cl_pad, qn=qn):
            ga = wid * ROWS_PER_W + 2 * r2
            gb = ga + 1

            @pl.when(r2 > 0)
            def _():
                drain_store(pack0, sem_s0)
                drain_store(pack1, sem_s1)

            ha = pltpu.async_copy(
                tok_hbm.at[pl.ds(ga * SEQ_PAD + off, cl_pad)],
                idx0.at[pl.ds(0, cl_pad)], sem_i0)
            hb = pltpu.async_copy(
                tok_hbm.at[pl.ds(gb * SEQ_PAD + off, cl_pad)],
                idx1.at[pl.ds(0, cl_pad)], sem_i1)

            ha.wait()
            hga = fire_gathers(idx0, rows0, sem_g0)
            hb.wait()
            hgb = fire_gathers(idx1, rows1, sem_g1)

            for h in hga:
                h.wait()
            add_pack(rows0, pack0)
            pltpu.async_copy(
                pack0.at[pl.ds(0, qn), :],
                out_hbm.at[ga, pl.ds(off // 2, qn), :], sem_s0)

            for h in hgb:
                h.wait()
            add_pack(rows1, pack1)
            pltpu.async_copy(
                pack1.at[pl.ds(0, qn), :],
                out_hbm.at[gb, pl.ds(off // 2, qn), :], sem_s1)
            return carry

        lax.fori_loop(0, ROWS_PER_W // 2, pair_body, 0)
        drain_store(pack0, sem_s0)
        drain_store(pack1, sem_s1)


def _unpack_body(pin, pout):
    x = pin[0]
    a = x[:, :DIM]
    b = x[:, DIM:]
    y = jnp.concatenate([a[:, None, :], b[:, None, :]], axis=1)
    pout[...] = y.reshape(1, 512, DIM)


_unpack = pl.pallas_call(
    _unpack_body,
    grid=(BATCH, 3),
    in_specs=[pl.BlockSpec((1, 256, 128), lambda b, c: (b, c, 0))],
    out_specs=pl.BlockSpec((1, 512, DIM), lambda b, c: (b, c, 0)),
    out_shape=jax.ShapeDtypeStruct((BATCH, SEQ, DIM), jnp.float32),
)


def kernel(tokens, table):
    pe = _pe_table()
    tok_flat = jnp.pad(tokens, ((0, 0), (0, SEQ_PAD - SEQ))).reshape(-1)
    packed = _seq_encode(tok_flat, pe, table)
    return _unpack(packed)


# speedup vs baseline: 2.5144x; 2.5144x over previous
import functools
import math

import jax
import jax.numpy as jnp
from jax import lax
from jax.experimental import pallas as pl
from jax.experimental.pallas import tpu as pltpu
from jax.experimental.pallas import tpu_sc as plsc

VOCAB = 100000
DIM = 64
SEQ = 1500
BATCH = 1024
SEQ_PAD = 1504

NC = 2
NS = 16
NW = NC * NS
ROWS_PER_W = BATCH // NW

CHUNKS = ((0, 512), (512, 512), (1024, 476))
CLMAX = 512
IGS = 128


def _pe_table():
    position = jnp.arange(SEQ, dtype=jnp.float32)[:, None]
    div_term = jnp.exp(
        jnp.arange(0, DIM, 2, dtype=jnp.float32) * (-(math.log(10000.0) / DIM))
    )
    ang = position * div_term
    pe = jnp.zeros((SEQ, DIM), dtype=jnp.float32)
    pe = pe.at[:, 0::2].set(jnp.sin(ang))
    pe = pe.at[:, 1::2].set(jnp.cos(ang))
    return pe


@functools.partial(
    pl.kernel,
    mesh=plsc.VectorSubcoreMesh(core_axis_name="c", subcore_axis_name="s"),
    out_type=jax.ShapeDtypeStruct((BATCH, SEQ, DIM), jnp.float32),
    scratch_types=(
        [pltpu.VMEM((CLMAX, DIM), jnp.float32)]
        + [pltpu.VMEM((CLMAX,), jnp.int32) for _ in range(4)]
        + [pltpu.VMEM((CLMAX, DIM), jnp.float32) for _ in range(2)]
        + [pltpu.SemaphoreType.DMA for _ in range(4)]
        + [pltpu.SemaphoreType.DMA for _ in range(2)]
        + [pltpu.SemaphoreType.DMA for _ in range(2)]
    ),
    compiler_params=pltpu.CompilerParams(use_tc_tiling_on_sc=False),
)
def _seq_encode(tok_hbm, pe_hbm, table_hbm, out_hbm, pe_v, *scratch):
    idx = scratch[0:4]
    rows = scratch[4:6]
    sem_i = scratch[6:10]
    sem_g = scratch[10:12]
    sem_s = scratch[12:14]
    wid = lax.axis_index("s") * NC + lax.axis_index("c")

    for off, cl in CHUNKS:
        cl_pad = -(-cl // 8) * 8
        n_g, rem = divmod(cl_pad, IGS)
        unroll = 8 if cl % 8 == 0 else 4

        pltpu.sync_copy(pe_hbm.at[pl.ds(off, cl), :], pe_v.at[pl.ds(0, cl), :])

        def fire_idx(g, b, off=off, cl_pad=cl_pad):
            pltpu.async_copy(
                tok_hbm.at[pl.ds(g * SEQ_PAD + off, cl_pad)],
                idx[b].at[pl.ds(0, cl_pad)], sem_i[b])

        def drain_idx(b, cl_pad=cl_pad):
            pltpu.make_async_copy(
                tok_hbm.at[pl.ds(0, cl_pad)],
                idx[b].at[pl.ds(0, cl_pad)], sem_i[b]).wait()

        def fire_gathers(b, rb, n_g=n_g, rem=rem):
            for j in range(n_g):
                pltpu.async_copy(
                    table_hbm.at[idx[b].at[pl.ds(j * IGS, IGS)]],
                    rows[rb].at[pl.ds(j * IGS, IGS), :], sem_g[rb])
            if rem:
                pltpu.async_copy(
                    table_hbm.at[idx[b].at[pl.ds(n_g * IGS, rem)]],
                    rows[rb].at[pl.ds(n_g * IGS, rem), :], sem_g[rb])

        def drain_gathers(rb, n_g=n_g, rem=rem):
            for j in range(n_g):
                pltpu.make_async_copy(
                    pe_hbm.at[pl.ds(0, IGS), :],
                    rows[rb].at[pl.ds(j * IGS, IGS), :], sem_g[rb]).wait()
            if rem:
                pltpu.make_async_copy(
                    pe_hbm.at[pl.ds(0, rem), :],
                    rows[rb].at[pl.ds(n_g * IGS, rem), :], sem_g[rb]).wait()

        def add_pe(rb, cl=cl, unroll=unroll):
            def add_body(i, c):
                for u in range(unroll):
                    p = i * unroll + u
                    for v in range(DIM // 16):
                        plsc.addupdate(
                            rows[rb].at[p, pl.ds(v * 16, 16)],
                            pe_v[p, pl.ds(v * 16, 16)],
                        )
                return c
            lax.fori_loop(0, cl // unroll, add_body, 0)

        def fire_store(g, rb, off=off, cl=cl):
            pltpu.async_copy(
                rows[rb].at[pl.ds(0, cl), :],
                out_hbm.at[g, pl.ds(off, cl), :], sem_s[rb])

        def drain_store(rb, off=off, cl=cl):
            pltpu.make_async_copy(
                rows[rb].at[pl.ds(0, cl), :],
                out_hbm.at[0, pl.ds(off, cl), :], sem_s[rb]).wait()

        def do_pair(p2, ia, ib, first):
            ga = wid * ROWS_PER_W + 2 * p2
            gb = ga + 1

            @pl.when(p2 > 0)
            def _():
                drain_store(0)
            drain_idx(ia)
            fire_gathers(ia, 0)

            @pl.when(p2 > 0)
            def _():
                drain_store(1)
            drain_idx(ib)
            fire_gathers(ib, 1)

            @pl.when(2 * p2 + 2 < ROWS_PER_W)
            def _():
                fire_idx(ga + 2, (ia + 2) % 4)
                fire_idx(gb + 2, (ib + 2) % 4)

            drain_gathers(0)
            add_pe(0)
            fire_store(ga, 0)

            drain_gathers(1)
            add_pe(1)
            fire_store(gb, 1)

        fire_idx(wid * ROWS_PER_W, 0)
        fire_idx(wid * ROWS_PER_W + 1, 1)

        def quad_body(i, carry):
            do_pair(2 * i, 0, 1, False)
            do_pair(2 * i + 1, 2, 3, False)
            return carry

        lax.fori_loop(0, ROWS_PER_W // 4, quad_body, 0)
        drain_store(0)
        drain_store(1)


def kernel(tokens, table):
    pe = _pe_table()
    tok_flat = jnp.pad(tokens, ((0, 0), (0, SEQ_PAD - SEQ))).reshape(-1)
    return _seq_encode(tok_flat, pe, table)


# speedup vs baseline: 2.5346x; 1.0081x over previous
import functools
import math

import jax
import jax.numpy as jnp
from jax import lax
from jax.experimental import pallas as pl
from jax.experimental.pallas import tpu as pltpu
from jax.experimental.pallas import tpu_sc as plsc

VOCAB = 100000
DIM = 64
SEQ = 1500
BATCH = 1024
SEQ_PAD = 1504

NC = 2
NS = 16
NW = NC * NS
ROWS_PER_W = BATCH // NW

CHUNKS = ((0, 512), (512, 512), (1024, 476))
CLMAX = 512
IGS = 128


def _pe_table():
    position = jnp.arange(SEQ, dtype=jnp.float32)[:, None]
    div_term = jnp.exp(
        jnp.arange(0, DIM, 2, dtype=jnp.float32) * (-(math.log(10000.0) / DIM))
    )
    ang = position * div_term
    pe = jnp.zeros((SEQ, DIM), dtype=jnp.float32)
    pe = pe.at[:, 0::2].set(jnp.sin(ang))
    pe = pe.at[:, 1::2].set(jnp.cos(ang))
    return pe


@functools.partial(
    pl.kernel,
    mesh=plsc.VectorSubcoreMesh(core_axis_name="c", subcore_axis_name="s"),
    out_type=jax.ShapeDtypeStruct((BATCH, SEQ, DIM), jnp.float32),
    scratch_types=(
        [pltpu.VMEM((CLMAX, DIM), jnp.float32)]
        + [pltpu.VMEM((CLMAX,), jnp.int32) for _ in range(4)]
        + [pltpu.VMEM((CLMAX, DIM), jnp.float32) for _ in range(2)]
        + [pltpu.SemaphoreType.DMA for _ in range(4)]
        + [pltpu.SemaphoreType.DMA for _ in range(8)]
        + [pltpu.SemaphoreType.DMA for _ in range(2)]
    ),
    compiler_params=pltpu.CompilerParams(use_tc_tiling_on_sc=False),
)
def _seq_encode(tok_hbm, pe_hbm, table_hbm, out_hbm, pe_v, *scratch):
    idx = scratch[0:4]
    rows = scratch[4:6]
    sem_i = scratch[6:10]
    sem_g = (scratch[10:14], scratch[14:18])
    sem_s = scratch[18:20]
    wid = lax.axis_index("s") * NC + lax.axis_index("c")

    for off, cl in CHUNKS:
        cl_pad = -(-cl // 8) * 8
        n_g, rem = divmod(cl_pad, IGS)
        unroll = 8 if cl % 8 == 0 else 4

        pltpu.sync_copy(pe_hbm.at[pl.ds(off, cl), :], pe_v.at[pl.ds(0, cl), :])

        def fire_idx(g, b, off=off, cl_pad=cl_pad):
            pltpu.async_copy(
                tok_hbm.at[pl.ds(g * SEQ_PAD + off, cl_pad)],
                idx[b].at[pl.ds(0, cl_pad)], sem_i[b])

        def drain_idx(b, cl_pad=cl_pad):
            pltpu.make_async_copy(
                tok_hbm.at[pl.ds(0, cl_pad)],
                idx[b].at[pl.ds(0, cl_pad)], sem_i[b]).wait()

        def fire_gathers(b, rb, n_g=n_g, rem=rem):
            for j in range(n_g):
                pltpu.async_copy(
                    table_hbm.at[idx[b].at[pl.ds(j * IGS, IGS)]],
                    rows[rb].at[pl.ds(j * IGS, IGS), :], sem_g[rb][j])
            if rem:
                pltpu.async_copy(
                    table_hbm.at[idx[b].at[pl.ds(n_g * IGS, rem)]],
                    rows[rb].at[pl.ds(n_g * IGS, rem), :], sem_g[rb][n_g])

        def drain_gather(rb, j, n_g=n_g, rem=rem):
            n = IGS if j < n_g else rem
            pltpu.make_async_copy(
                pe_hbm.at[pl.ds(0, n), :],
                rows[rb].at[pl.ds(j * IGS, n), :], sem_g[rb][j]).wait()

        def add_block(rb, base, n):
            unroll = 8 if n % 8 == 0 else 4

            def add_body(i, c):
                for u in range(unroll):
                    p = base + i * unroll + u
                    for v in range(DIM // 16):
                        plsc.addupdate(
                            rows[rb].at[p, pl.ds(v * 16, 16)],
                            pe_v[p, pl.ds(v * 16, 16)],
                        )
                return c
            lax.fori_loop(0, n // unroll, add_body, 0)

        blocks = []
        done = 0
        for j in range(n_g + (1 if rem else 0)):
            n = min(IGS, cl - done)
            blocks.append((j, done, n))
            done += n

        def drain_add(rb):
            for j, base, n in blocks:
                drain_gather(rb, j)
                add_block(rb, base, n)

        def fire_store(g, rb, off=off, cl=cl):
            pltpu.async_copy(
                rows[rb].at[pl.ds(0, cl), :],
                out_hbm.at[g, pl.ds(off, cl), :], sem_s[rb])

        def drain_store(rb, off=off, cl=cl):
            pltpu.make_async_copy(
                rows[rb].at[pl.ds(0, cl), :],
                out_hbm.at[0, pl.ds(off, cl), :], sem_s[rb]).wait()

        def do_pair(p2, ia, ib, first):
            ga = wid * ROWS_PER_W + 2 * p2
            gb = ga + 1

            @pl.when(p2 > 0)
            def _():
                drain_store(0)
            drain_idx(ia)
            fire_gathers(ia, 0)

            @pl.when(p2 > 0)
            def _():
                drain_store(1)
            drain_idx(ib)
            fire_gathers(ib, 1)

            @pl.when(2 * p2 + 2 < ROWS_PER_W)
            def _():
                fire_idx(ga + 2, (ia + 2) % 4)
                fire_idx(gb + 2, (ib + 2) % 4)

            drain_add(0)
            fire_store(ga, 0)

            drain_add(1)
            fire_store(gb, 1)

        fire_idx(wid * ROWS_PER_W, 0)
        fire_idx(wid * ROWS_PER_W + 1, 1)

        def quad_body(i, carry):
            do_pair(2 * i, 0, 1, False)
            do_pair(2 * i + 1, 2, 3, False)
            return carry

        lax.fori_loop(0, ROWS_PER_W // 4, quad_body, 0)
        drain_store(0)
        drain_store(1)


def kernel(tokens, table):
    pe = _pe_table()
    tok_flat = jnp.pad(tokens, ((0, 0), (0, SEQ_PAD - SEQ))).reshape(-1)
    return _seq_encode(tok_flat, pe, table)
